# baseline (speedup 1.0000x reference)
import functools

import jax
import jax.numpy as jnp
from jax import lax
from jax.experimental import pallas as pl
from jax.experimental.pallas import tpu as pltpu, tpu_sc as plsc

NUM_ACTIONS = 100000
D = 128
B = 16384
HALF = B // 2
NC, NS = 2, 16
PAIR_ROWS = HALF // NS

G0 = 224
Z0 = 240
GCHUNKS = {0: (112, 112), 1: (96, 96, 96)}
ZSHARE = {0: Z0, 1: PAIR_ROWS - Z0}
ZBLK = 16

MAXCHUNK = 112

_mesh = plsc.VectorSubcoreMesh(core_axis_name="c", subcore_axis_name="s")


@functools.partial(
    pl.kernel,
    out_type=jax.ShapeDtypeStruct((B, D), jnp.float32),
    mesh=_mesh,
    scratch_types=[
        pltpu.VMEM((PAIR_ROWS,), jnp.int32),
        [pltpu.VMEM((MAXCHUNK, D), jnp.float32)] * 3,
        pltpu.VMEM((ZBLK, D), jnp.float32),
        [pltpu.SemaphoreType.DMA] * 3,
        pltpu.SemaphoreType.DMA,
    ],
)
def _embed_gather(idx_hbm, table_hbm, out_hbm,
                  idxv, rowsb, zbuf, sems, semz):
    c = lax.axis_index("c")
    s = lax.axis_index("s")

    def _core_plan(core):
        gsizes = GCHUNKS[core]
        g_rows = sum(gsizes)
        z_rows = ZSHARE[core]
        gbase = s * PAIR_ROWS + (0 if core == 0 else G0)
        zbase = s * PAIR_ROWS + (0 if core == 0 else Z0)

        iload = pltpu.async_copy(
            idx_hbm.at[pl.ds(HALF + gbase, g_rows)],
            idxv.at[pl.ds(0, g_rows)], sems[0])

        iload.wait()
        gathers, off = [], 0
        for j, sz in enumerate(gsizes):
            gathers.append(pltpu.async_copy(
                table_hbm.at[idxv.at[pl.ds(off, sz)]],
                rowsb[j].at[pl.ds(0, sz)], sems[j]))
            off += sz

        z16 = jnp.zeros((16,), jnp.float32)

        def _zfill(i, carry):
            for k in range(D // 16):
                zbuf[i, pl.ds(k * 16, 16)] = z16
            return carry

        lax.fori_loop(0, ZBLK, _zfill, 0)
        zwrites = [
            pltpu.async_copy(
                zbuf, out_hbm.at[pl.ds(zbase + z * ZBLK, ZBLK)], semz)
            for z in range(z_rows // ZBLK)
        ]

        rwrites, off = [], 0
        for j, sz in enumerate(gsizes):
            gathers[j].wait()
            rwrites.append(pltpu.async_copy(
                rowsb[j].at[pl.ds(0, sz)],
                out_hbm.at[pl.ds(HALF + gbase + off, sz)], sems[j]))
            off += sz
        for cp in zwrites:
            cp.wait()
        for cp in rwrites:
            cp.wait()

    @pl.when(c == 0)
    def _():
        _core_plan(0)

    @pl.when(c == 1)
    def _():
        _core_plan(1)


def kernel(input, action_embedding):
    idx_all = input.reshape(B)
    out = _embed_gather(idx_all, action_embedding)
    return out[None]


# speedup vs baseline: 1.0117x; 1.0117x over previous
import functools

import jax
import jax.numpy as jnp
from jax import lax
from jax.experimental import pallas as pl
from jax.experimental.pallas import tpu as pltpu, tpu_sc as plsc

NUM_ACTIONS = 100000
D = 128
B = 16384
HALF = B // 2
NC, NS = 2, 16
PAIR_ROWS = HALF // NS
ROWS_PER_W = 256
CHUNK = 128
NCHUNK = ROWS_PER_W // CHUNK
ZBLK = 16
Z0 = 224

_mesh = plsc.VectorSubcoreMesh(core_axis_name="c", subcore_axis_name="s")


@functools.partial(
    pl.kernel,
    out_type=jax.ShapeDtypeStruct((B, D), jnp.float32),
    mesh=_mesh,
    scratch_types=[
        pltpu.VMEM((ROWS_PER_W,), jnp.int32),
        [pltpu.VMEM((CHUNK, D), jnp.float32)] * NCHUNK,
        pltpu.VMEM((ZBLK, D), jnp.float32),
        [pltpu.SemaphoreType.DMA] * NCHUNK,
        pltpu.SemaphoreType.DMA,
    ],
)
def _embed_gather(idx_hbm, table_hbm, out_hbm,
                  idxv, rowsb, zbuf, sems, semz):
    c = lax.axis_index("c")
    s = lax.axis_index("s")
    base = (s * NC + c) * ROWS_PER_W

    iload = pltpu.async_copy(
        idx_hbm.at[pl.ds(HALF + base, ROWS_PER_W)], idxv, sems[0])

    iload.wait()
    gathers = []
    for j in range(NCHUNK):
        gathers.append(pltpu.async_copy(
            table_hbm.at[idxv.at[pl.ds(j * CHUNK, CHUNK)]], rowsb[j], sems[j]))

    z16 = jnp.zeros((16,), jnp.float32)

    def _zfill(i, carry):
        for k in range(D // 16):
            zbuf[i, pl.ds(k * 16, 16)] = z16
        return carry

    lax.fori_loop(0, ZBLK, _zfill, 0)

    nblk = jnp.where(c == 0, Z0 // ZBLK, (PAIR_ROWS - Z0) // ZBLK)
    zbase = s * PAIR_ROWS + jnp.where(c == 0, 0, Z0)

    def _zissue(i, carry):
        pltpu.async_copy(
            zbuf, out_hbm.at[pl.ds(zbase + i * ZBLK, ZBLK)], semz)
        return carry

    lax.fori_loop(0, nblk, _zissue, 0)

    rwrites = []
    for j in range(NCHUNK):
        gathers[j].wait()
        rwrites.append(pltpu.async_copy(
            rowsb[j], out_hbm.at[pl.ds(HALF + base + j * CHUNK, CHUNK)],
            sems[j]))

    def _zdrain(i, carry):
        pltpu.make_async_copy(
            zbuf, out_hbm.at[pl.ds(zbase + i * ZBLK, ZBLK)], semz).wait()
        return carry

    lax.fori_loop(0, nblk, _zdrain, 0)
    for cp in rwrites:
        cp.wait()


def kernel(input, action_embedding):
    idx_all = input.reshape(B)
    out = _embed_gather(idx_all, action_embedding)
    return out[None]


# speedup vs baseline: 1.0209x; 1.0090x over previous
import functools

import jax
import jax.numpy as jnp
from jax import lax
from jax.experimental import pallas as pl
from jax.experimental.pallas import tpu as pltpu, tpu_sc as plsc

NUM_ACTIONS = 100000
D = 128
B = 16384
HALF = B // 2
NC, NS = 2, 16
PAIR_ROWS = HALF // NS
ROWS_PER_W = 256
CHUNK = 128
NCHUNK = ROWS_PER_W // CHUNK
ZBLK = 16
Z0 = 208

_mesh = plsc.VectorSubcoreMesh(core_axis_name="c", subcore_axis_name="s")


@functools.partial(
    pl.kernel,
    out_type=jax.ShapeDtypeStruct((B, D), jnp.float32),
    mesh=_mesh,
    scratch_types=[
        pltpu.VMEM((ROWS_PER_W,), jnp.int32),
        [pltpu.VMEM((CHUNK, D), jnp.float32)] * NCHUNK,
        pltpu.VMEM((ZBLK, D), jnp.float32),
        [pltpu.SemaphoreType.DMA] * NCHUNK,
        pltpu.SemaphoreType.DMA,
    ],
)
def _embed_gather(idx_hbm, table_hbm, out_hbm,
                  idxv, rowsb, zbuf, sems, semz):
    c = lax.axis_index("c")
    s = lax.axis_index("s")
    base = (s * NC + c) * ROWS_PER_W

    iload = pltpu.async_copy(
        idx_hbm.at[pl.ds(HALF + base, ROWS_PER_W)], idxv, sems[0])

    z16 = jnp.zeros((16,), jnp.float32)

    def _zfill(i, carry):
        for k in range(D // 16):
            zbuf[i, pl.ds(k * 16, 16)] = z16
        return carry

    lax.fori_loop(0, ZBLK, _zfill, 0)

    nblk = jnp.where(c == 0, Z0 // ZBLK, (PAIR_ROWS - Z0) // ZBLK)
    zbase = s * PAIR_ROWS + jnp.where(c == 0, 0, Z0)

    def _zissue(i, carry):
        pltpu.async_copy(
            zbuf, out_hbm.at[pl.ds(zbase + i * ZBLK, ZBLK)], semz)
        return carry

    lax.fori_loop(0, nblk, _zissue, 0)

    iload.wait()
    gathers = []
    for j in range(NCHUNK):
        gathers.append(pltpu.async_copy(
            table_hbm.at[idxv.at[pl.ds(j * CHUNK, CHUNK)]], rowsb[j], sems[j]))

    rwrites = []
    for j in range(NCHUNK):
        gathers[j].wait()
        rwrites.append(pltpu.async_copy(
            rowsb[j], out_hbm.at[pl.ds(HALF + base + j * CHUNK, CHUNK)],
            sems[j]))

    def _zdrain(i, carry):
        pltpu.make_async_copy(
            zbuf, out_hbm.at[pl.ds(zbase + i * ZBLK, ZBLK)], semz).wait()
        return carry

    lax.fori_loop(0, nblk, _zdrain, 0)
    for cp in rwrites:
        cp.wait()


def kernel(input, action_embedding):
    idx_all = input.reshape(B)
    out = _embed_gather(idx_all, action_embedding)
    return out[None]
